# baseline (speedup 1.0000x reference)
import functools

import numpy as np

import jax
import jax.numpy as jnp
from jax import lax
from jax.experimental import pallas as pl
from jax.experimental.pallas import tpu as pltpu
from jax.experimental.pallas import tpu_sc as plsc

N = 10000
E = 640000
F_IN = 119
H = 300
HP = 384
QW = HP // 4
B = 10
R = 1264

CH = 112
SB = 16
NCHUNK = -(-E // (CH * 16 * SB)) * (16 * SB)
EPAD = NCHUNK * CH
CH_PER_TILE = NCHUNK // 16
NSB = CH_PER_TILE // SB
NP = 10112
AGG_ROWS = NP

SIGMA = np.arange(HP).reshape(-1, 16, 2).transpose(0, 2, 1).reshape(-1)


def _make_sc_pass(p):
    mesh = plsc.VectorSubcoreMesh(core_axis_name="c", subcore_axis_name="s",
                                  num_cores=2)

    @functools.partial(
        pl.kernel,
        out_type=jax.ShapeDtypeStruct((2, NP, QW), jnp.float32),
        mesh=mesh,
        compiler_params=pltpu.CompilerParams(use_tc_tiling_on_sc=False,
                                             needs_layout_passes=False),
        scratch_types=[
            pltpu.VMEM((2, SB, CH), jnp.int32),
            pltpu.VMEM((2, SB, CH), jnp.int32),
            pltpu.VMEM((CH, QW), jnp.bfloat16),
            pltpu.VMEM((CH, QW), jnp.bfloat16),
            pltpu.VMEM((CH, QW), jnp.float32),
            pltpu.VMEM((CH, QW), jnp.float32),
            pltpu.VMEM_SHARED((NP, QW), jnp.bfloat16),
            pltpu.VMEM_SHARED((AGG_ROWS, QW), jnp.float32),
            pltpu.SemaphoreType.DMA,
            pltpu.SemaphoreType.DMA,
            pltpu.SemaphoreType.DMA,
            pltpu.SemaphoreType.DMA,
            pltpu.SemaphoreType.DMA,
        ],
    )
    def sc_pass(y_hbm, src_hbm, dst_hbm, zeros_hbm, out_hbm,
                src_v, dst_v, grow_0, grow_1, frow_0, frow_1, tbl, agg,
                gs0, gs1, ss0, ss1, isem):
        grows = (grow_0, grow_1)
        frows = (frow_0, frow_1)
        gsem = (gs0, gs1)
        ssem = (ss0, ss1)
        c = lax.axis_index("c")
        s = lax.axis_index("s")
        q = 2 * c + p

        rt = pl.multiple_of(s * (NP // 16), 8)
        co = pl.multiple_of(QW * q, 8)
        pltpu.sync_copy(y_hbm.at[pl.ds(rt, NP // 16), pl.ds(co, QW)],
                        tbl.at[pl.ds(rt, NP // 16)])

        pltpu.sync_copy(zeros_hbm.at[pl.ds(rt, NP // 16)],
                        agg.at[pl.ds(rt, NP // 16)])
        plsc.subcore_barrier()

        def convert(gb, fb):
            def cbody(i, carry):
                e0 = pl.multiple_of(i * 8, 8)
                for r in range(8):
                    for g in range(QW // 32):
                        x = gb[e0 + r, pl.ds(32 * g, 32)]
                        a, b_ = plsc.unpack(x, format=plsc.PackFormat.INTERLEAVED)
                        fb[e0 + r, pl.ds(32 * g, 16)] = a
                        fb[e0 + r, pl.ds(32 * g + 16, 16)] = b_
                return carry

            lax.fori_loop(0, CH // 8, cbody, 0)

        def idx_row0(k):
            return pl.multiple_of(s * CH_PER_TILE + k * SB, SB)

        def fire_idx(k, par):
            pltpu.async_copy(src_hbm.at[pl.ds(idx_row0(k), SB)],
                             src_v.at[par], isem)
            pltpu.async_copy(dst_hbm.at[pl.ds(idx_row0(k), SB)],
                             dst_v.at[par], isem)

        fire_idx(0, 0)

        def sbody(k, carry):
            par = lax.rem(k, 2)
            pltpu.make_async_copy(src_hbm.at[pl.ds(idx_row0(k), SB)],
                                  src_v.at[par], isem).wait()
            pltpu.make_async_copy(dst_hbm.at[pl.ds(idx_row0(k), SB)],
                                  dst_v.at[par], isem).wait()

            @pl.when(k + 1 < NSB)
            def _():
                fire_idx(k + 1, 1 - par)

            sv = src_v.at[par]
            dv = dst_v.at[par]
            gcp, scp = {}, {}
            gcp[0] = pltpu.async_copy(tbl.at[sv.at[0]], grows[0], gsem[0])
            for j in range(SB):
                if j + 1 < SB:
                    gcp[j + 1] = pltpu.async_copy(
                        tbl.at[sv.at[j + 1]], grows[(j + 1) % 2],
                        gsem[(j + 1) % 2])
                gcp[j].wait()
                if j >= 2:
                    scp[j - 2].wait()
                convert(grows[j % 2], frows[j % 2])
                scp[j] = pltpu.async_copy(
                    frows[j % 2], agg.at[dv.at[j]], ssem[j % 2], add=True)
            scp[SB - 2].wait()
            scp[SB - 1].wait()
            return carry

        lax.fori_loop(0, NSB, sbody, 0)
        plsc.subcore_barrier()

        pltpu.sync_copy(agg.at[pl.ds(rt, NP // 16)],
                        out_hbm.at[c].at[pl.ds(rt, NP // 16)])

    return sc_pass


_sc_pass0 = _make_sc_pass(0)
_sc_pass1 = _make_sc_pass(1)


def _dot(a, b):
    return jnp.dot(a, b, preferred_element_type=jnp.float32)


def _split4(y, out):
    out[...] = y.astype(jnp.bfloat16)


def _combine(p0, p1, w4):
    return (_dot(p0[0], w4[0][...]) + _dot(p1[0], w4[1][...]) +
            _dot(p0[1], w4[2][...]) + _dot(p1[1], w4[3][...]))


_W4_SPECS = [pl.BlockSpec((QW, H), lambda i: (0, 0)) for _ in range(4)]
_P_SPEC = pl.BlockSpec((2, R, QW), lambda i: (0, i, 0))
_Y_SPEC = pl.BlockSpec((R, HP), lambda i: (i, 0))
_Y_SHAPE = jax.ShapeDtypeStruct((NP, HP), jnp.bfloat16)


def _tc_lift(node_feats, W_lift, b_lift, W1p, b1p):
    def body(nf, wl, bl, w1, b1, out):
        x = _dot(nf[...], wl[...]) + bl[...]
        y = jnp.maximum(_dot(x, w1[...]) + b1[...], 0.0)
        _split4(y, out)

    return pl.pallas_call(
        body,
        grid=(NP // R,),
        in_specs=[
            pl.BlockSpec((R, F_IN), lambda i: (i, 0)),
            pl.BlockSpec((F_IN, H), lambda i: (0, 0)),
            pl.BlockSpec((1, H), lambda i: (0, 0)),
            pl.BlockSpec((H, HP), lambda i: (0, 0)),
            pl.BlockSpec((1, HP), lambda i: (0, 0)),
        ],
        out_specs=_Y_SPEC,
        out_shape=_Y_SHAPE,
    )(node_feats, W_lift, b_lift, W1p, b1p)


def _tc_layer(p0, p1, W2q, b2, W1p, b1p):
    def body(a0, a1, w2a, w2b, w2c, w2d, b2r, w1, b1r, out):
        x = jnp.maximum(_combine(a0, a1, (w2a, w2b, w2c, w2d)) + b2r[...], 0.0)
        y = jnp.maximum(_dot(x, w1[...]) + b1r[...], 0.0)
        _split4(y, out)

    return pl.pallas_call(
        body,
        grid=(NP // R,),
        in_specs=[_P_SPEC, _P_SPEC] + _W4_SPECS + [
            pl.BlockSpec((1, H), lambda i: (0, 0)),
            pl.BlockSpec((H, HP), lambda i: (0, 0)),
            pl.BlockSpec((1, HP), lambda i: (0, 0)),
        ],
        out_specs=_Y_SPEC,
        out_shape=_Y_SHAPE,
    )(p0, p1, *W2q, b2, W1p, b1p)


def _tc_final(p0, p1, W2q, b2, Wro, bro, gid):
    def body(a0, a1, w2a, w2b, w2c, w2d, b2r, wro, bror, g, out):
        x = jnp.maximum(_combine(a0, a1, (w2a, w2b, w2c, w2d)) + b2r[...], 0.0)
        logits = _dot(x, wro[...]) + bror[...]
        row = (lax.broadcasted_iota(jnp.int32, (R, 1), 0)
               + pl.program_id(0) * R)
        logits = jnp.where(row < N, logits, 0.0)
        oh = (g[...] == lax.broadcasted_iota(jnp.int32, (1, 16), 1))
        part = lax.dot_general(oh.astype(jnp.float32), logits,
                               (((0,), (0,)), ((), ())),
                               preferred_element_type=jnp.float32)

        @pl.when(pl.program_id(0) == 0)
        def _():
            out[...] = jnp.zeros_like(out)

        out[...] += part

    return pl.pallas_call(
        body,
        grid=(NP // R,),
        in_specs=[_P_SPEC, _P_SPEC] + _W4_SPECS + [
            pl.BlockSpec((1, H), lambda i: (0, 0)),
            pl.BlockSpec((H, 128), lambda i: (0, 0)),
            pl.BlockSpec((1, 128), lambda i: (0, 0)),
            pl.BlockSpec((R, 1), lambda i: (i, 0)),
        ],
        out_specs=pl.BlockSpec((16, 128), lambda i: (0, 0)),
        out_shape=jax.ShapeDtypeStruct((16, 128), jnp.float32),
    )(p0, p1, *W2q, b2, Wro, bro, gid)


def kernel(node_feats, edge_index, graph_ids, W_lift, b_lift, W1a, b1a,
           W2a, b2a, W1b, b1b, W2b, b2b, W1c, b1c, W2c, b2c, W_ro, b_ro):
    f32 = jnp.float32
    srcs = jnp.concatenate(
        [edge_index[0], jnp.zeros((EPAD - E,), jnp.int32)]).reshape(NCHUNK, CH)
    dsts = jnp.concatenate(
        [edge_index[1], jnp.full((EPAD - E,), N, jnp.int32)]).reshape(NCHUNK, CH)
    zeros = jnp.zeros((AGG_ROWS, QW), f32)

    def msg_w(W1, b1):
        return (jnp.pad(W1, ((0, 0), (0, HP - H))),
                jnp.pad(b1, (0, HP - H)).reshape(1, HP))

    def upd_w(W2):
        Wp = jnp.pad(W2, ((0, HP - H), (0, 0)))[SIGMA]
        return tuple(Wp[q * QW:(q + 1) * QW] for q in range(4))

    W1a_p, b1a_p = msg_w(W1a, b1a)
    W1b_p, b1b_p = msg_w(W1b, b1b)
    W1c_p, b1c_p = msg_w(W1c, b1c)
    W2a_q = upd_w(W2a)
    W2b_q = upd_w(W2b)
    W2c_q = upd_w(W2c)
    Wro_p = jnp.pad(W_ro, ((0, 0), (0, 128 - W_ro.shape[1])))
    bro_p = jnp.pad(b_ro, (0, 128 - b_ro.shape[0])).reshape(1, 128)

    def sc_layer(y):
        a0 = _sc_pass0(y, srcs, dsts, zeros)
        a1 = _sc_pass1(y, srcs, dsts, zeros)
        return a0, a1

    ya = _tc_lift(node_feats, W_lift, b_lift.reshape(1, H), W1a_p, b1a_p)
    a0, a1 = sc_layer(ya)
    yb = _tc_layer(a0, a1, W2a_q, b2a.reshape(1, H), W1b_p, b1b_p)
    b0, b1 = sc_layer(yb)
    yc = _tc_layer(b0, b1, W2b_q, b2b.reshape(1, H), W1c_p, b1c_p)
    c0, c1 = sc_layer(yc)
    out = _tc_final(c0, c1, W2c_q, b2c.reshape(1, H), Wro_p, bro_p,
                    graph_ids.reshape(N, 1))
    return out[:B, :2]


# speedup vs baseline: 1.5691x; 1.5691x over previous
import functools

import jax
import jax.numpy as jnp
from jax import lax
from jax.experimental import pallas as pl
from jax.experimental.pallas import tpu as pltpu
from jax.experimental.pallas import tpu_sc as plsc

N = 10000
E = 640000
F_IN = 119
H = 300
HP = 320
QW = HP // 4
B = 10
R = 1000

CH = 96
SB = 16
NCHUNK = -(-E // (CH * 16 * SB)) * (16 * SB)
EPAD = NCHUNK * CH
CH_PER_TILE = NCHUNK // 16
NSB = CH_PER_TILE // SB
AGG_ROWS = 10240
NP = 10112
WB_CHUNKS = NP // CH


def _make_sc_pass(p):
    mesh = plsc.VectorSubcoreMesh(core_axis_name="c", subcore_axis_name="s",
                                  num_cores=2)

    @functools.partial(
        pl.kernel,
        out_type=jax.ShapeDtypeStruct((2, NP, QW), jnp.float32),
        mesh=mesh,
        compiler_params=pltpu.CompilerParams(use_tc_tiling_on_sc=False),
        scratch_types=[
            pltpu.VMEM((2, SB, CH), jnp.int32),
            pltpu.VMEM((2, SB, CH), jnp.int32),
            pltpu.VMEM((CH, QW), jnp.float32),
            pltpu.VMEM((CH, QW), jnp.float32),
            pltpu.VMEM((CH, QW), jnp.float32),
            pltpu.VMEM_SHARED((NP, QW), jnp.float32),
            pltpu.VMEM_SHARED((AGG_ROWS, QW), jnp.float32),
            pltpu.SemaphoreType.DMA,
            pltpu.SemaphoreType.DMA,
            pltpu.SemaphoreType.DMA,
            pltpu.SemaphoreType.DMA,
            pltpu.SemaphoreType.DMA,
            pltpu.SemaphoreType.DMA,
            pltpu.SemaphoreType.DMA,
        ],
    )
    def sc_pass(y_hbm, src_hbm, dst_hbm, zeros_hbm, out_hbm,
                src_v, dst_v, rows_0, rows_1, rows_2, tbl, agg,
                gs0, gs1, gs2, ss0, ss1, ss2, isem):
        rows = (rows_0, rows_1, rows_2)
        gsem = (gs0, gs1, gs2)
        ssem = (ss0, ss1, ss2)
        c = lax.axis_index("c")
        s = lax.axis_index("s")
        q = 2 * c + p

        rt = pl.multiple_of(s * (NP // 16), 8)
        pltpu.sync_copy(y_hbm.at[q].at[pl.ds(rt, NP // 16)],
                        tbl.at[pl.ds(rt, NP // 16)])

        rz = pl.multiple_of(s * (AGG_ROWS // 16), 8)
        pltpu.sync_copy(zeros_hbm.at[pl.ds(rz, AGG_ROWS // 16)],
                        agg.at[pl.ds(rz, AGG_ROWS // 16)])
        plsc.subcore_barrier()

        def idx_row0(k):
            return pl.multiple_of(s * CH_PER_TILE + k * SB, SB)

        def fire_idx(k, par):
            pltpu.async_copy(src_hbm.at[pl.ds(idx_row0(k), SB)],
                             src_v.at[par], isem)
            pltpu.async_copy(dst_hbm.at[pl.ds(idx_row0(k), SB)],
                             dst_v.at[par], isem)

        fire_idx(0, 0)

        def sbody(k, carry):
            par = lax.rem(k, 2)
            pltpu.make_async_copy(src_hbm.at[pl.ds(idx_row0(k), SB)],
                                  src_v.at[par], isem).wait()
            pltpu.make_async_copy(dst_hbm.at[pl.ds(idx_row0(k), SB)],
                                  dst_v.at[par], isem).wait()

            @pl.when(k + 1 < NSB)
            def _():
                fire_idx(k + 1, 1 - par)

            sv = src_v.at[par]
            dv = dst_v.at[par]
            nb = len(rows)
            gcp, scp = {}, {}
            for j in range(nb - 1):
                gcp[j] = pltpu.async_copy(tbl.at[sv.at[j]], rows[j], gsem[j])
            for j in range(SB):
                nxt = j + nb - 1
                if nxt < SB:
                    if nxt - nb >= 0:
                        scp[nxt - nb].wait()
                    gcp[nxt] = pltpu.async_copy(
                        tbl.at[sv.at[nxt]], rows[nxt % nb], gsem[nxt % nb])
                gcp[j].wait()
                scp[j] = pltpu.async_copy(
                    rows[j % nb], agg.at[dv.at[j]], ssem[j % nb], add=True)
            for j in range(SB - nb, SB):
                scp[j].wait()
            return carry

        lax.fori_loop(0, NSB, sbody, 0)
        plsc.subcore_barrier()

        pltpu.sync_copy(agg.at[pl.ds(rt, NP // 16)],
                        out_hbm.at[c].at[pl.ds(rt, NP // 16)])

    return sc_pass


_sc_pass0 = _make_sc_pass(0)
_sc_pass1 = _make_sc_pass(1)


def _dot(a, b):
    return jnp.dot(a, b, preferred_element_type=jnp.float32)


def _split4(y, out):
    for q_ in range(4):
        out[q_] = y[:, q_ * QW:(q_ + 1) * QW]


def _combine(p0, p1, w4):
    return (_dot(p0[0], w4[0][...]) + _dot(p1[0], w4[1][...]) +
            _dot(p0[1], w4[2][...]) + _dot(p1[1], w4[3][...]))


_W4_SPECS = [pl.BlockSpec((QW, H), lambda i: (0, 0)) for _ in range(4)]
_P_SPEC = pl.BlockSpec((2, R, QW), lambda i: (0, i, 0))
_Y_SPEC = pl.BlockSpec((4, R, QW), lambda i: (0, i, 0))
_Y_SHAPE = jax.ShapeDtypeStruct((4, NP, QW), jnp.float32)


def _tc_lift(node_feats, W_lift, b_lift, W1p, b1p):
    def body(nf, wl, bl, w1, b1, out):
        x = _dot(nf[...], wl[...]) + bl[...]
        y = jnp.maximum(_dot(x, w1[...]) + b1[...], 0.0)
        _split4(y, out)

    return pl.pallas_call(
        body,
        grid=(N // R,),
        in_specs=[
            pl.BlockSpec((R, F_IN), lambda i: (i, 0)),
            pl.BlockSpec((F_IN, H), lambda i: (0, 0)),
            pl.BlockSpec((1, H), lambda i: (0, 0)),
            pl.BlockSpec((H, HP), lambda i: (0, 0)),
            pl.BlockSpec((1, HP), lambda i: (0, 0)),
        ],
        out_specs=_Y_SPEC,
        out_shape=_Y_SHAPE,
    )(node_feats, W_lift, b_lift, W1p, b1p)


def _tc_layer(p0, p1, W2q, b2, W1p, b1p):
    def body(a0, a1, w2a, w2b, w2c, w2d, b2r, w1, b1r, out):
        x = jnp.maximum(_combine(a0, a1, (w2a, w2b, w2c, w2d)) + b2r[...], 0.0)
        y = jnp.maximum(_dot(x, w1[...]) + b1r[...], 0.0)
        _split4(y, out)

    return pl.pallas_call(
        body,
        grid=(N // R,),
        in_specs=[_P_SPEC, _P_SPEC] + _W4_SPECS + [
            pl.BlockSpec((1, H), lambda i: (0, 0)),
            pl.BlockSpec((H, HP), lambda i: (0, 0)),
            pl.BlockSpec((1, HP), lambda i: (0, 0)),
        ],
        out_specs=_Y_SPEC,
        out_shape=_Y_SHAPE,
    )(p0, p1, *W2q, b2, W1p, b1p)


def _tc_final(p0, p1, W2q, b2, Wro, bro, gid):
    def body(a0, a1, w2a, w2b, w2c, w2d, b2r, wro, bror, g, out):
        x = jnp.maximum(_combine(a0, a1, (w2a, w2b, w2c, w2d)) + b2r[...], 0.0)
        logits = _dot(x, wro[...]) + bror[...]
        oh = (g[...] == lax.broadcasted_iota(jnp.int32, (1, 16), 1))
        part = lax.dot_general(oh.astype(jnp.float32), logits,
                               (((0,), (0,)), ((), ())),
                               preferred_element_type=jnp.float32)

        @pl.when(pl.program_id(0) == 0)
        def _():
            out[...] = jnp.zeros_like(out)

        out[...] += part

    return pl.pallas_call(
        body,
        grid=(N // R,),
        in_specs=[_P_SPEC, _P_SPEC] + _W4_SPECS + [
            pl.BlockSpec((1, H), lambda i: (0, 0)),
            pl.BlockSpec((H, 128), lambda i: (0, 0)),
            pl.BlockSpec((1, 128), lambda i: (0, 0)),
            pl.BlockSpec((R, 1), lambda i: (i, 0)),
        ],
        out_specs=pl.BlockSpec((16, 128), lambda i: (0, 0)),
        out_shape=jax.ShapeDtypeStruct((16, 128), jnp.float32),
    )(p0, p1, *W2q, b2, Wro, bro, gid)


def kernel(node_feats, edge_index, graph_ids, W_lift, b_lift, W1a, b1a,
           W2a, b2a, W1b, b1b, W2b, b2b, W1c, b1c, W2c, b2c, W_ro, b_ro):
    f32 = jnp.float32
    srcs = jnp.concatenate(
        [edge_index[0], jnp.zeros((EPAD - E,), jnp.int32)]).reshape(NCHUNK, CH)
    dsts = jnp.concatenate(
        [edge_index[1], jnp.full((EPAD - E,), N, jnp.int32)]).reshape(NCHUNK, CH)
    zeros = jnp.zeros((AGG_ROWS, QW), f32)

    def msg_w(W1, b1):
        return (jnp.pad(W1, ((0, 0), (0, HP - H))),
                jnp.pad(b1, (0, HP - H)).reshape(1, HP))

    def upd_w(W2):
        Wp = jnp.pad(W2, ((0, HP - H), (0, 0)))
        return tuple(Wp[q * QW:(q + 1) * QW] for q in range(4))

    W1a_p, b1a_p = msg_w(W1a, b1a)
    W1b_p, b1b_p = msg_w(W1b, b1b)
    W1c_p, b1c_p = msg_w(W1c, b1c)
    W2a_q = upd_w(W2a)
    W2b_q = upd_w(W2b)
    W2c_q = upd_w(W2c)
    Wro_p = jnp.pad(W_ro, ((0, 0), (0, 128 - W_ro.shape[1])))
    bro_p = jnp.pad(b_ro, (0, 128 - b_ro.shape[0])).reshape(1, 128)

    def sc_layer(y):
        a0 = _sc_pass0(y, srcs, dsts, zeros)
        a1 = _sc_pass1(y, srcs, dsts, zeros)
        return a0, a1

    ya = _tc_lift(node_feats, W_lift, b_lift.reshape(1, H), W1a_p, b1a_p)
    a0, a1 = sc_layer(ya)
    yb = _tc_layer(a0, a1, W2a_q, b2a.reshape(1, H), W1b_p, b1b_p)
    b0, b1 = sc_layer(yb)
    yc = _tc_layer(b0, b1, W2b_q, b2b.reshape(1, H), W1c_p, b1c_p)
    c0, c1 = sc_layer(yc)
    out = _tc_final(c0, c1, W2c_q, b2c.reshape(1, H), Wro_p, bro_p,
                    graph_ids.reshape(N, 1))
    return out[:B, :2]


# speedup vs baseline: 1.5985x; 1.0188x over previous
import functools

import jax
import jax.numpy as jnp
from jax import lax
from jax.experimental import pallas as pl
from jax.experimental.pallas import tpu as pltpu
from jax.experimental.pallas import tpu_sc as plsc

N = 10000
E = 640000
F_IN = 119
H = 300
HP = 320
QW = HP // 4
B = 10
R = 1000

CH = 128
SB = 16
NCHUNK = -(-E // (CH * 16 * SB)) * (16 * SB)
EPAD = NCHUNK * CH
CH_PER_TILE = NCHUNK // 16
NSB = CH_PER_TILE // SB
AGG_ROWS = 10240
NP = 10112
WB_CHUNKS = NP // CH


def _make_sc_pass(p):
    mesh = plsc.VectorSubcoreMesh(core_axis_name="c", subcore_axis_name="s",
                                  num_cores=2)

    @functools.partial(
        pl.kernel,
        out_type=jax.ShapeDtypeStruct((2, NP, QW), jnp.float32),
        mesh=mesh,
        compiler_params=pltpu.CompilerParams(use_tc_tiling_on_sc=False),
        scratch_types=[
            pltpu.VMEM((2, SB, CH), jnp.int32),
            pltpu.VMEM((2, SB, CH), jnp.int32),
            pltpu.VMEM((CH, QW), jnp.float32),
            pltpu.VMEM((CH, QW), jnp.float32),
            pltpu.VMEM_SHARED((NP, QW), jnp.float32),
            pltpu.VMEM_SHARED((AGG_ROWS, QW), jnp.float32),
            pltpu.SemaphoreType.DMA,
            pltpu.SemaphoreType.DMA,
            pltpu.SemaphoreType.DMA,
            pltpu.SemaphoreType.DMA,
            pltpu.SemaphoreType.DMA,
        ],
    )
    def sc_pass(y_hbm, src_hbm, dst_hbm, zeros_hbm, out_hbm,
                src_v, dst_v, rows_0, rows_1, tbl, agg, gs0, gs1, ss0, ss1,
                isem):
        rows = (rows_0, rows_1)
        gsem = (gs0, gs1)
        ssem = (ss0, ss1)
        c = lax.axis_index("c")
        s = lax.axis_index("s")
        q = 2 * c + p

        rt = pl.multiple_of(s * (NP // 16), 8)
        pltpu.sync_copy(y_hbm.at[q].at[pl.ds(rt, NP // 16)],
                        tbl.at[pl.ds(rt, NP // 16)])

        rz = pl.multiple_of(s * (AGG_ROWS // 16), 8)
        pltpu.sync_copy(zeros_hbm.at[pl.ds(rz, AGG_ROWS // 16)],
                        agg.at[pl.ds(rz, AGG_ROWS // 16)])
        plsc.subcore_barrier()

        def idx_row0(k):
            return pl.multiple_of(s * CH_PER_TILE + k * SB, SB)

        def fire_idx(k, par):
            pltpu.async_copy(src_hbm.at[pl.ds(idx_row0(k), SB)],
                             src_v.at[par], isem)
            pltpu.async_copy(dst_hbm.at[pl.ds(idx_row0(k), SB)],
                             dst_v.at[par], isem)

        fire_idx(0, 0)

        def sbody(k, carry):
            par = lax.rem(k, 2)
            pltpu.make_async_copy(src_hbm.at[pl.ds(idx_row0(k), SB)],
                                  src_v.at[par], isem).wait()
            pltpu.make_async_copy(dst_hbm.at[pl.ds(idx_row0(k), SB)],
                                  dst_v.at[par], isem).wait()

            @pl.when(k + 1 < NSB)
            def _():
                fire_idx(k + 1, 1 - par)

            sv = src_v.at[par]
            dv = dst_v.at[par]
            gcp, scp = {}, {}
            gcp[0] = pltpu.async_copy(tbl.at[sv.at[0]], rows[0], gsem[0])
            for j in range(SB):
                if j + 1 < SB:
                    if j >= 1:
                        scp[j - 1].wait()
                    gcp[j + 1] = pltpu.async_copy(
                        tbl.at[sv.at[j + 1]], rows[(j + 1) % 2],
                        gsem[(j + 1) % 2])
                gcp[j].wait()
                scp[j] = pltpu.async_copy(
                    rows[j % 2], agg.at[dv.at[j]], ssem[j % 2], add=True)
            scp[SB - 2].wait()
            scp[SB - 1].wait()
            return carry

        lax.fori_loop(0, NSB, sbody, 0)
        plsc.subcore_barrier()

        pltpu.sync_copy(agg.at[pl.ds(rt, NP // 16)],
                        out_hbm.at[c].at[pl.ds(rt, NP // 16)])

    return sc_pass


_sc_pass0 = _make_sc_pass(0)
_sc_pass1 = _make_sc_pass(1)


def _dot(a, b):
    return jnp.dot(a, b, preferred_element_type=jnp.float32)


def _split4(y, out):
    for q_ in range(4):
        out[q_] = y[:, q_ * QW:(q_ + 1) * QW]


def _combine(p0, p1, w4):
    return (_dot(p0[0], w4[0][...]) + _dot(p1[0], w4[1][...]) +
            _dot(p0[1], w4[2][...]) + _dot(p1[1], w4[3][...]))


_W4_SPECS = [pl.BlockSpec((QW, H), lambda i: (0, 0)) for _ in range(4)]
_P_SPEC = pl.BlockSpec((2, R, QW), lambda i: (0, i, 0))
_Y_SPEC = pl.BlockSpec((4, R, QW), lambda i: (0, i, 0))
_Y_SHAPE = jax.ShapeDtypeStruct((4, NP, QW), jnp.float32)


def _tc_lift(node_feats, W_lift, b_lift, W1p, b1p):
    def body(nf, wl, bl, w1, b1, out):
        x = _dot(nf[...], wl[...]) + bl[...]
        y = jnp.maximum(_dot(x, w1[...]) + b1[...], 0.0)
        _split4(y, out)

    return pl.pallas_call(
        body,
        grid=(N // R,),
        in_specs=[
            pl.BlockSpec((R, F_IN), lambda i: (i, 0)),
            pl.BlockSpec((F_IN, H), lambda i: (0, 0)),
            pl.BlockSpec((1, H), lambda i: (0, 0)),
            pl.BlockSpec((H, HP), lambda i: (0, 0)),
            pl.BlockSpec((1, HP), lambda i: (0, 0)),
        ],
        out_specs=_Y_SPEC,
        out_shape=_Y_SHAPE,
    )(node_feats, W_lift, b_lift, W1p, b1p)


def _tc_layer(p0, p1, W2q, b2, W1p, b1p):
    def body(a0, a1, w2a, w2b, w2c, w2d, b2r, w1, b1r, out):
        x = jnp.maximum(_combine(a0, a1, (w2a, w2b, w2c, w2d)) + b2r[...], 0.0)
        y = jnp.maximum(_dot(x, w1[...]) + b1r[...], 0.0)
        _split4(y, out)

    return pl.pallas_call(
        body,
        grid=(N // R,),
        in_specs=[_P_SPEC, _P_SPEC] + _W4_SPECS + [
            pl.BlockSpec((1, H), lambda i: (0, 0)),
            pl.BlockSpec((H, HP), lambda i: (0, 0)),
            pl.BlockSpec((1, HP), lambda i: (0, 0)),
        ],
        out_specs=_Y_SPEC,
        out_shape=_Y_SHAPE,
    )(p0, p1, *W2q, b2, W1p, b1p)


def _tc_final(p0, p1, W2q, b2, Wro, bro, gid):
    def body(a0, a1, w2a, w2b, w2c, w2d, b2r, wro, bror, g, out):
        x = jnp.maximum(_combine(a0, a1, (w2a, w2b, w2c, w2d)) + b2r[...], 0.0)
        logits = _dot(x, wro[...]) + bror[...]
        oh = (g[...] == lax.broadcasted_iota(jnp.int32, (1, 16), 1))
        part = lax.dot_general(oh.astype(jnp.float32), logits,
                               (((0,), (0,)), ((), ())),
                               preferred_element_type=jnp.float32)

        @pl.when(pl.program_id(0) == 0)
        def _():
            out[...] = jnp.zeros_like(out)

        out[...] += part

    return pl.pallas_call(
        body,
        grid=(N // R,),
        in_specs=[_P_SPEC, _P_SPEC] + _W4_SPECS + [
            pl.BlockSpec((1, H), lambda i: (0, 0)),
            pl.BlockSpec((H, 128), lambda i: (0, 0)),
            pl.BlockSpec((1, 128), lambda i: (0, 0)),
            pl.BlockSpec((R, 1), lambda i: (i, 0)),
        ],
        out_specs=pl.BlockSpec((16, 128), lambda i: (0, 0)),
        out_shape=jax.ShapeDtypeStruct((16, 128), jnp.float32),
    )(p0, p1, *W2q, b2, Wro, bro, gid)


def kernel(node_feats, edge_index, graph_ids, W_lift, b_lift, W1a, b1a,
           W2a, b2a, W1b, b1b, W2b, b2b, W1c, b1c, W2c, b2c, W_ro, b_ro):
    f32 = jnp.float32
    srcs = jnp.concatenate(
        [edge_index[0], jnp.zeros((EPAD - E,), jnp.int32)]).reshape(NCHUNK, CH)
    dsts = jnp.concatenate(
        [edge_index[1], jnp.full((EPAD - E,), N, jnp.int32)]).reshape(NCHUNK, CH)
    zeros = jnp.zeros((AGG_ROWS, QW), f32)

    def msg_w(W1, b1):
        return (jnp.pad(W1, ((0, 0), (0, HP - H))),
                jnp.pad(b1, (0, HP - H)).reshape(1, HP))

    def upd_w(W2):
        Wp = jnp.pad(W2, ((0, HP - H), (0, 0)))
        return tuple(Wp[q * QW:(q + 1) * QW] for q in range(4))

    W1a_p, b1a_p = msg_w(W1a, b1a)
    W1b_p, b1b_p = msg_w(W1b, b1b)
    W1c_p, b1c_p = msg_w(W1c, b1c)
    W2a_q = upd_w(W2a)
    W2b_q = upd_w(W2b)
    W2c_q = upd_w(W2c)
    Wro_p = jnp.pad(W_ro, ((0, 0), (0, 128 - W_ro.shape[1])))
    bro_p = jnp.pad(b_ro, (0, 128 - b_ro.shape[0])).reshape(1, 128)

    def sc_layer(y):
        a0 = _sc_pass0(y, srcs, dsts, zeros)
        a1 = _sc_pass1(y, srcs, dsts, zeros)
        return a0, a1

    ya = _tc_lift(node_feats, W_lift, b_lift.reshape(1, H), W1a_p, b1a_p)
    a0, a1 = sc_layer(ya)
    yb = _tc_layer(a0, a1, W2a_q, b2a.reshape(1, H), W1b_p, b1b_p)
    b0, b1 = sc_layer(yb)
    yc = _tc_layer(b0, b1, W2b_q, b2b.reshape(1, H), W1c_p, b1c_p)
    c0, c1 = sc_layer(yc)
    out = _tc_final(c0, c1, W2c_q, b2c.reshape(1, H), Wro_p, bro_p,
                    graph_ids.reshape(N, 1))
    return out[:B, :2]


# speedup vs baseline: 1.5998x; 1.0008x over previous
import functools

import jax
import jax.numpy as jnp
from jax import lax
from jax.experimental import pallas as pl
from jax.experimental.pallas import tpu as pltpu
from jax.experimental.pallas import tpu_sc as plsc

N = 10000
E = 640000
F_IN = 119
H = 300
HP = 320
QW = HP // 4
B = 10
R = 1000

CH = 128
SB = 16
NCHUNK = -(-E // (CH * 16 * SB)) * (16 * SB)
EPAD = NCHUNK * CH
CH_PER_TILE = NCHUNK // 16
NSB = CH_PER_TILE // SB
AGG_ROWS = 10240
NP = 10112
WB_CHUNKS = NP // CH


@functools.lru_cache(maxsize=None)
def _make_sc_pass(p):
    mesh = plsc.VectorSubcoreMesh(core_axis_name="c", subcore_axis_name="s",
                                  num_cores=2)

    @functools.partial(
        pl.kernel,
        out_type=jax.ShapeDtypeStruct((2, NP, QW), jnp.float32),
        mesh=mesh,
        compiler_params=pltpu.CompilerParams(use_tc_tiling_on_sc=False),
        scratch_types=[
            pltpu.VMEM((2, SB, CH), jnp.int32),
            pltpu.VMEM((2, SB, CH), jnp.int32),
            pltpu.VMEM((CH, QW), jnp.float32),
            pltpu.VMEM((CH, QW), jnp.float32),
            pltpu.VMEM_SHARED((NP, QW), jnp.float32),
            pltpu.VMEM_SHARED((AGG_ROWS, QW), jnp.float32),
            pltpu.SemaphoreType.DMA,
            pltpu.SemaphoreType.DMA,
            pltpu.SemaphoreType.DMA,
            pltpu.SemaphoreType.DMA,
            pltpu.SemaphoreType.DMA,
        ],
    )
    def sc_pass(y_hbm, src_hbm, dst_hbm, zeros_hbm, out_hbm,
                src_v, dst_v, rows_0, rows_1, tbl, agg, gs0, gs1, ss0, ss1,
                isem):
        rows = (rows_0, rows_1)
        gsem = (gs0, gs1)
        ssem = (ss0, ss1)
        c = lax.axis_index("c")
        s = lax.axis_index("s")
        q = 2 * c + p

        rt = pl.multiple_of(s * (NP // 16), 8)
        pltpu.sync_copy(y_hbm.at[q].at[pl.ds(rt, NP // 16)],
                        tbl.at[pl.ds(rt, NP // 16)])

        rz = pl.multiple_of(s * (AGG_ROWS // 16), 8)
        pltpu.sync_copy(zeros_hbm.at[pl.ds(rz, AGG_ROWS // 16)],
                        agg.at[pl.ds(rz, AGG_ROWS // 16)])
        plsc.subcore_barrier()

        def idx_row0(k):
            return pl.multiple_of(s * CH_PER_TILE + k * SB, SB)

        def fire_idx(k, par):
            pltpu.async_copy(src_hbm.at[pl.ds(idx_row0(k), SB)],
                             src_v.at[par], isem)
            pltpu.async_copy(dst_hbm.at[pl.ds(idx_row0(k), SB)],
                             dst_v.at[par], isem)

        fire_idx(0, 0)

        def sbody(k, carry):
            par = lax.rem(k, 2)
            pltpu.make_async_copy(src_hbm.at[pl.ds(idx_row0(k), SB)],
                                  src_v.at[par], isem).wait()
            pltpu.make_async_copy(dst_hbm.at[pl.ds(idx_row0(k), SB)],
                                  dst_v.at[par], isem).wait()

            @pl.when(k + 1 < NSB)
            def _():
                fire_idx(k + 1, 1 - par)

            sv = src_v.at[par]
            dv = dst_v.at[par]
            gcp, scp = {}, {}
            gcp[0] = pltpu.async_copy(tbl.at[sv.at[0]], rows[0], gsem[0])
            for j in range(SB):
                if j + 1 < SB:
                    if j >= 1:
                        scp[j - 1].wait()
                    gcp[j + 1] = pltpu.async_copy(
                        tbl.at[sv.at[j + 1]], rows[(j + 1) % 2],
                        gsem[(j + 1) % 2])
                gcp[j].wait()
                scp[j] = pltpu.async_copy(
                    rows[j % 2], agg.at[dv.at[j]], ssem[j % 2], add=True)
            scp[SB - 2].wait()
            scp[SB - 1].wait()
            return carry

        lax.fori_loop(0, NSB, sbody, 0)
        plsc.subcore_barrier()

        pltpu.sync_copy(agg.at[pl.ds(rt, NP // 16)],
                        out_hbm.at[c].at[pl.ds(rt, NP // 16)])

    return sc_pass


def _dot(a, b):
    return jnp.dot(a, b, preferred_element_type=jnp.float32)


def _split4(y, out):
    for q_ in range(4):
        out[q_] = y[:, q_ * QW:(q_ + 1) * QW]


def _combine(p0, p1, w4):
    return (_dot(p0[0], w4[0][...]) + _dot(p1[0], w4[1][...]) +
            _dot(p0[1], w4[2][...]) + _dot(p1[1], w4[3][...]))


_W4_SPECS = [pl.BlockSpec((QW, H), lambda i: (0, 0)) for _ in range(4)]
_P_SPEC = pl.BlockSpec((2, R, QW), lambda i: (0, i, 0))
_Y_SPEC = pl.BlockSpec((4, R, QW), lambda i: (0, i, 0))
_Y_SHAPE = jax.ShapeDtypeStruct((4, NP, QW), jnp.float32)


def _tc_lift(node_feats, W_lift, b_lift, W1p, b1p):
    def body(nf, wl, bl, w1, b1, out):
        x = _dot(nf[...], wl[...]) + bl[...]
        y = jnp.maximum(_dot(x, w1[...]) + b1[...], 0.0)
        _split4(y, out)

    return pl.pallas_call(
        body,
        grid=(N // R,),
        in_specs=[
            pl.BlockSpec((R, F_IN), lambda i: (i, 0)),
            pl.BlockSpec((F_IN, H), lambda i: (0, 0)),
            pl.BlockSpec((1, H), lambda i: (0, 0)),
            pl.BlockSpec((H, HP), lambda i: (0, 0)),
            pl.BlockSpec((1, HP), lambda i: (0, 0)),
        ],
        out_specs=_Y_SPEC,
        out_shape=_Y_SHAPE,
    )(node_feats, W_lift, b_lift, W1p, b1p)


def _tc_layer(p0, p1, W2q, b2, W1p, b1p):
    def body(a0, a1, w2a, w2b, w2c, w2d, b2r, w1, b1r, out):
        x = jnp.maximum(_combine(a0, a1, (w2a, w2b, w2c, w2d)) + b2r[...], 0.0)
        y = jnp.maximum(_dot(x, w1[...]) + b1r[...], 0.0)
        _split4(y, out)

    return pl.pallas_call(
        body,
        grid=(N // R,),
        in_specs=[_P_SPEC, _P_SPEC] + _W4_SPECS + [
            pl.BlockSpec((1, H), lambda i: (0, 0)),
            pl.BlockSpec((H, HP), lambda i: (0, 0)),
            pl.BlockSpec((1, HP), lambda i: (0, 0)),
        ],
        out_specs=_Y_SPEC,
        out_shape=_Y_SHAPE,
    )(p0, p1, *W2q, b2, W1p, b1p)


def _tc_final(p0, p1, W2q, b2, Wro, bro, gid):
    def body(a0, a1, w2a, w2b, w2c, w2d, b2r, wro, bror, g, out):
        x = jnp.maximum(_combine(a0, a1, (w2a, w2b, w2c, w2d)) + b2r[...], 0.0)
        logits = _dot(x, wro[...]) + bror[...]
        oh = (g[...] == lax.broadcasted_iota(jnp.int32, (1, 16), 1))
        part = lax.dot_general(oh.astype(jnp.float32), logits,
                               (((0,), (0,)), ((), ())),
                               preferred_element_type=jnp.float32)

        @pl.when(pl.program_id(0) == 0)
        def _():
            out[...] = jnp.zeros_like(out)

        out[...] += part

    return pl.pallas_call(
        body,
        grid=(N // R,),
        in_specs=[_P_SPEC, _P_SPEC] + _W4_SPECS + [
            pl.BlockSpec((1, H), lambda i: (0, 0)),
            pl.BlockSpec((H, 128), lambda i: (0, 0)),
            pl.BlockSpec((1, 128), lambda i: (0, 0)),
            pl.BlockSpec((R, 1), lambda i: (i, 0)),
        ],
        out_specs=pl.BlockSpec((16, 128), lambda i: (0, 0)),
        out_shape=jax.ShapeDtypeStruct((16, 128), jnp.float32),
    )(p0, p1, *W2q, b2, Wro, bro, gid)


def kernel(node_feats, edge_index, graph_ids, W_lift, b_lift, W1a, b1a,
           W2a, b2a, W1b, b1b, W2b, b2b, W1c, b1c, W2c, b2c, W_ro, b_ro):
    f32 = jnp.float32
    srcs = jnp.concatenate(
        [edge_index[0], jnp.zeros((EPAD - E,), jnp.int32)]).reshape(NCHUNK, CH)
    dsts = jnp.concatenate(
        [edge_index[1], jnp.full((EPAD - E,), N, jnp.int32)]).reshape(NCHUNK, CH)
    zeros = jnp.zeros((AGG_ROWS, QW), f32)

    def msg_w(W1, b1):
        return (jnp.pad(W1, ((0, 0), (0, HP - H))),
                jnp.pad(b1, (0, HP - H)).reshape(1, HP))

    def upd_w(W2):
        Wp = jnp.pad(W2, ((0, HP - H), (0, 0)))
        return tuple(Wp[q * QW:(q + 1) * QW] for q in range(4))

    W1a_p, b1a_p = msg_w(W1a, b1a)
    W1b_p, b1b_p = msg_w(W1b, b1b)
    W1c_p, b1c_p = msg_w(W1c, b1c)
    W2a_q = upd_w(W2a)
    W2b_q = upd_w(W2b)
    W2c_q = upd_w(W2c)
    Wro_p = jnp.pad(W_ro, ((0, 0), (0, 128 - W_ro.shape[1])))
    bro_p = jnp.pad(b_ro, (0, 128 - b_ro.shape[0])).reshape(1, 128)

    def sc_layer(y):
        a0 = _make_sc_pass(0)(y, srcs, dsts, zeros)
        a1 = _make_sc_pass(1)(y, srcs, dsts, zeros)
        return a0, a1

    ya = _tc_lift(node_feats, W_lift, b_lift.reshape(1, H), W1a_p, b1a_p)
    a0, a1 = sc_layer(ya)
    yb = _tc_layer(a0, a1, W2a_q, b2a.reshape(1, H), W1b_p, b1b_p)
    b0, b1 = sc_layer(yb)
    yc = _tc_layer(b0, b1, W2b_q, b2b.reshape(1, H), W1c_p, b1c_p)
    c0, c1 = sc_layer(yc)
    out = _tc_final(c0, c1, W2c_q, b2c.reshape(1, H), Wro_p, bro_p,
                    graph_ids.reshape(N, 1))
    return out[:B, :2]
